# baseline (speedup 1.0000x reference)
import functools

import jax
import jax.numpy as jnp
from jax import lax
from jax.experimental import pallas as pl
from jax.experimental.pallas import tpu as pltpu
from jax.experimental.pallas import tpu_sc as plsc

N = 10000
NPAD = 10112
E = 320000
EP = 327680
EROWS = 2560
DEG_CPT = EROWS // 16
ECH = 64
ACH = EP // ECH
CPT = ACH // 32
NPASS = 4
PCH = CPT // NPASS
PIR = PCH // 2
RPT = NPAD // 16
D = 128
DEGW = 8

_MESH = plsc.VectorSubcoreMesh(core_axis_name="c", subcore_axis_name="s")


@functools.partial(
    pl.kernel,
    mesh=_MESH,
    out_type=jax.ShapeDtypeStruct((2, 16, NPAD), jnp.float32),
    compiler_params=pltpu.CompilerParams(needs_layout_passes=False),
    scratch_types=[
        pltpu.VMEM((DEG_CPT, 128), jnp.int32),
        pltpu.VMEM((NPAD,), jnp.float32),
    ],
)
def _sc_degrees(edges_hbm, out_hbm, idx_v, hist_v):
    c = lax.axis_index("c")
    s = lax.axis_index("s")
    pltpu.sync_copy(edges_hbm.at[c, pl.ds(s * DEG_CPT, DEG_CPT)], idx_v)

    def zbody(i, carry):
        hist_v[pl.ds(i * 16, 16)] = jnp.zeros((16,), jnp.float32)
        return carry

    lax.fori_loop(0, NPAD // 16, zbody, 0)
    ones16 = jnp.ones((16,), jnp.float32)

    def body(j, carry):
        for k in range(8):
            idx16 = idx_v[j, pl.ds(k * 16, 16)]
            plsc.addupdate_scatter(hist_v, [idx16], ones16)
        return carry

    lax.fori_loop(0, DEG_CPT, body, 0)
    pltpu.sync_copy(hist_v, out_hbm.at[c, s])


@functools.partial(
    pl.kernel,
    mesh=_MESH,
    out_type=jax.ShapeDtypeStruct((2, NPAD, D), jnp.float32),
    scratch_types=[
        pltpu.VMEM((PIR, 2, ECH), jnp.int32),
        pltpu.VMEM((PIR, 2, ECH), jnp.int32),
        pltpu.VMEM((ECH, D), jnp.float32),
        pltpu.VMEM((ECH, D), jnp.float32),
        pltpu.VMEM((ECH, D), jnp.float32),
        pltpu.VMEM((ECH, D), jnp.float32),
        pltpu.VMEM_SHARED((NPAD, D), jnp.float32),
        pltpu.SemaphoreType.DMA,
        pltpu.SemaphoreType.DMA,
        pltpu.SemaphoreType.DMA,
        pltpu.SemaphoreType.DMA,
        pltpu.SemaphoreType.DMA,
        pltpu.SemaphoreType.DMA,
        pltpu.SemaphoreType.DMA,
        pltpu.SemaphoreType.DMA,
    ],
)
def _sc_aggregate(h_hbm, edges_hbm, out_hbm, src_v, dst_v,
                  rows0_v, rows1_v, rows2_v, rows3_v, acc_sh,
                  g0, g1, g2, g3, s0, s1, s2, s3):
    c = lax.axis_index("c")
    s = lax.axis_index("s")
    wid = c * 16 + s
    bufs = (rows0_v, rows1_v, rows2_v, rows3_v)
    gsems = (g0, g1, g2, g3)
    ssems = (s0, s1, s2, s3)

    def _gather(chunk, b):
        pltpu.async_copy(h_hbm.at[src_v.at[chunk // 2, chunk % 2]],
                         bufs[b], gsems[b])

    def _gather_wait(b):
        pltpu.make_async_copy(h_hbm.at[src_v.at[0, 0]], bufs[b],
                              gsems[b]).wait()

    def _scatter(chunk, b):
        pltpu.async_copy(bufs[b], acc_sh.at[dst_v.at[chunk // 2, chunk % 2]],
                         ssems[b], add=True)

    def _scatter_wait(b):
        pltpu.make_async_copy(bufs[b], acc_sh.at[dst_v.at[0, 0]],
                              ssems[b]).wait()

    zeros16 = jnp.zeros((16,), jnp.float32)

    def zbody(i, carry):
        for k in range(8):
            rows0_v[i, pl.ds(k * 16, 16)] = zeros16
        return carry

    lax.fori_loop(0, ECH, zbody, 0)
    for k in range(RPT // ECH):
        pltpu.sync_copy(rows0_v, acc_sh.at[pl.ds(s * RPT + k * ECH, ECH)])
    rem = RPT - (RPT // ECH) * ECH
    pltpu.sync_copy(rows0_v.at[pl.ds(0, rem)],
                    acc_sh.at[pl.ds(s * RPT + (RPT // ECH) * ECH, rem)])
    plsc.subcore_barrier()

    for p in range(NPASS):
        base = wid * (CPT // 2) + p * PIR
        pltpu.sync_copy(edges_hbm.at[0, pl.ds(base, PIR)], src_v)
        pltpu.sync_copy(edges_hbm.at[1, pl.ds(base, PIR)], dst_v)

        _gather(0, 0)
        _gather(1, 1)

        def body(i, carry):
            for b in range(4):
                k = i * 4 + b
                _gather_wait(b)
                _scatter(k, b)
                b2 = (b + 2) % 4
                kp = k + 2

                @pl.when(k >= 2)
                def _():
                    _scatter_wait(b2)

                @pl.when(kp < PCH)
                def _():
                    _gather(kp, b2)

            return carry

        lax.fori_loop(0, PCH // 4, body, 0)
        _scatter_wait(2)
        _scatter_wait(3)

    plsc.subcore_barrier()

    pltpu.sync_copy(acc_sh.at[pl.ds(s * RPT, RPT)],
                    out_hbm.at[c, pl.ds(s * RPT, RPT)])


def _norm(deg):
    return jnp.where(deg > 0, lax.rsqrt(jnp.maximum(deg, 1.0)), 0.0)


def _tc_h1_body(hist_ref, x_ref, w1_ref, norms_ref, h1_ref):
    deg = jnp.sum(hist_ref[...], axis=1)
    norms = _norm(deg)
    norms_ref[...] = norms
    ns = norms[0, :N].reshape(N, 1)
    h1_ref[:N] = jnp.dot(x_ref[...] * ns, w1_ref[...],
                         preferred_element_type=jnp.float32)
    h1_ref[N:] = jnp.zeros((NPAD - N, D), jnp.float32)


def _tc2_body(part_ref, norms_ref, b1_ref, w2_ref, h2_ref):
    nd = norms_ref[1].reshape(NPAD, 1)
    ns = norms_ref[0].reshape(NPAD, 1)
    agg = part_ref[0] + part_ref[1]
    h1 = jnp.maximum(agg * nd + b1_ref[...], 0.0)
    h2_ref[...] = jnp.dot(h1 * ns, w2_ref[...],
                          preferred_element_type=jnp.float32)


def _tc3_body(part_ref, norms_ref, b2_ref, wp_ref, bp_ref, seg_ref, cam_ref):
    nd = norms_ref[1].reshape(NPAD, 1)
    agg = part_ref[0] + part_ref[1]
    hidden = jnp.maximum(agg * nd + b2_ref[...], 0.0)[:N]
    wp = wp_ref[...]
    cam_ref[...] = lax.dot_general(wp, hidden, (((1,), (1,)), ((), ())),
                                   preferred_element_type=jnp.float32)
    hg = jnp.sum(hidden, axis=0, keepdims=True) * (1.0 / N)
    seg_ref[...] = lax.dot_general(hg, wp, (((1,), (1,)), ((), ())),
                                   preferred_element_type=jnp.float32) + bp_ref[...]


def _tc_call(body, out_shape, *args):
    return pl.pallas_call(body, out_shape=out_shape)(*args)


def kernel(features, edge_index, is_training, W1, b1, W2, b2, Wp, bp):
    del is_training
    e = edge_index.astype(jnp.int32)
    spread = N + jnp.arange(EP - E, dtype=jnp.int32) % (NPAD - N)
    pad = jnp.stack([spread, spread])
    e = jnp.concatenate([e, pad], axis=1).reshape(2, EROWS, 128)
    e4 = e.reshape(2, EROWS, 2, ECH)

    hist = _sc_degrees(e)
    norms, h1 = pl.pallas_call(
        _tc_h1_body,
        out_shape=(jax.ShapeDtypeStruct((2, NPAD), jnp.float32),
                   jax.ShapeDtypeStruct((NPAD, D), jnp.float32)),
    )(hist, features, W1)

    part1 = _sc_aggregate(h1, e4)
    h2 = _tc_call(_tc2_body, jax.ShapeDtypeStruct((NPAD, D), jnp.float32),
                  part1, norms, b1.reshape(1, D), W2)
    part2 = _sc_aggregate(h2, e4)
    seg, cam = pl.pallas_call(
        _tc3_body,
        out_shape=(jax.ShapeDtypeStruct((1, 4), jnp.float32),
                   jax.ShapeDtypeStruct((4, N), jnp.float32)),
    )(part2, norms, b2.reshape(1, D), Wp, bp.reshape(1, 4))
    return seg, cam


# speedup vs baseline: 1.1284x; 1.1284x over previous
import functools

import jax
import jax.numpy as jnp
from jax import lax
from jax.experimental import pallas as pl
from jax.experimental.pallas import tpu as pltpu
from jax.experimental.pallas import tpu_sc as plsc

N = 10000
NPAD = 10112
E = 320000
EP = 327680
EROWS = 2560
CPT = EROWS // 32
PCH = CPT // 2
DEG_CPT = EROWS // 16
RPT = NPAD // 16
D = 128
DEGW = 8

_MESH = plsc.VectorSubcoreMesh(core_axis_name="c", subcore_axis_name="s")


@functools.partial(
    pl.kernel,
    mesh=_MESH,
    out_type=jax.ShapeDtypeStruct((2, 16, NPAD), jnp.float32),
    compiler_params=pltpu.CompilerParams(needs_layout_passes=False),
    scratch_types=[
        pltpu.VMEM((DEG_CPT, 128), jnp.int32),
        pltpu.VMEM((NPAD,), jnp.float32),
    ],
)
def _sc_degrees(edges_hbm, out_hbm, idx_v, hist_v):
    c = lax.axis_index("c")
    s = lax.axis_index("s")
    pltpu.sync_copy(edges_hbm.at[c, pl.ds(s * DEG_CPT, DEG_CPT)], idx_v)

    def zbody(i, carry):
        hist_v[pl.ds(i * 16, 16)] = jnp.zeros((16,), jnp.float32)
        return carry

    lax.fori_loop(0, NPAD // 16, zbody, 0)
    ones16 = jnp.ones((16,), jnp.float32)

    def body(j, carry):
        for k in range(8):
            idx16 = idx_v[j, pl.ds(k * 16, 16)]
            plsc.addupdate_scatter(hist_v, [idx16], ones16)
        return carry

    lax.fori_loop(0, DEG_CPT, body, 0)
    pltpu.sync_copy(hist_v, out_hbm.at[c, s])


@functools.partial(
    pl.kernel,
    mesh=_MESH,
    out_type=jax.ShapeDtypeStruct((2, NPAD, D), jnp.float32),
    scratch_types=[
        pltpu.VMEM((PCH, 128), jnp.int32),
        pltpu.VMEM((PCH, 128), jnp.int32),
        pltpu.VMEM((128, D), jnp.float32),
        pltpu.VMEM((128, D), jnp.float32),
        pltpu.VMEM_SHARED((NPAD, D), jnp.float32),
        pltpu.SemaphoreType.DMA,
        pltpu.SemaphoreType.DMA,
    ],
)
def _sc_aggregate(h_hbm, edges_hbm, out_hbm, src_v, dst_v,
                  rows0_v, rows1_v, acc_sh, sem0, sem1):
    c = lax.axis_index("c")
    s = lax.axis_index("s")
    wid = c * 16 + s
    bufs = (rows0_v, rows1_v)
    sems = (sem0, sem1)

    zeros16 = jnp.zeros((16,), jnp.float32)

    def zbody(i, carry):
        for k in range(8):
            rows0_v[i, pl.ds(k * 16, 16)] = zeros16
        return carry

    lax.fori_loop(0, 128, zbody, 0)
    for k in range(4):
        pltpu.sync_copy(rows0_v, acc_sh.at[pl.ds(s * RPT + k * 128, 128)])
    pltpu.sync_copy(rows0_v.at[pl.ds(0, RPT - 512)],
                    acc_sh.at[pl.ds(s * RPT + 512, RPT - 512)])
    plsc.subcore_barrier()

    for p in range(2):
        base = wid * CPT + p * PCH
        pltpu.sync_copy(edges_hbm.at[0, pl.ds(base, PCH)], src_v)
        pltpu.sync_copy(edges_hbm.at[1, pl.ds(base, PCH)], dst_v)

        pltpu.async_copy(h_hbm.at[src_v.at[0]], rows0_v, sem0)
        pltpu.async_copy(h_hbm.at[src_v.at[1]], rows1_v, sem1)

        def body(i, carry):
            for b in range(2):
                chunk = i * 2 + b
                pltpu.make_async_copy(h_hbm.at[src_v.at[chunk]], bufs[b],
                                      sems[b]).wait()
                pltpu.sync_copy(bufs[b], acc_sh.at[dst_v.at[chunk]], add=True)
                nxt = chunk + 2

                @pl.when(nxt < PCH)
                def _():
                    pltpu.async_copy(h_hbm.at[src_v.at[nxt]], bufs[b], sems[b])

            return carry

        lax.fori_loop(0, PCH // 2, body, 0)

    plsc.subcore_barrier()

    pltpu.sync_copy(acc_sh.at[pl.ds(s * RPT, RPT)],
                    out_hbm.at[c, pl.ds(s * RPT, RPT)])


def _norm(deg):
    return jnp.where(deg > 0, lax.rsqrt(jnp.maximum(deg, 1.0)), 0.0)


def _tc_h1_body(hist_ref, x_ref, w1_ref, norms_ref, h1_ref):
    deg = jnp.sum(hist_ref[...], axis=1)
    norms = _norm(deg)
    norms_ref[...] = norms
    ns = norms[0, :N].reshape(N, 1)
    h1_ref[:N] = jnp.dot(x_ref[...] * ns, w1_ref[...],
                         preferred_element_type=jnp.float32)
    h1_ref[N:] = jnp.zeros((NPAD - N, D), jnp.float32)


def _tc2_body(part_ref, norms_ref, b1_ref, w2_ref, h2_ref):
    nd = norms_ref[1].reshape(NPAD, 1)
    ns = norms_ref[0].reshape(NPAD, 1)
    agg = part_ref[0] + part_ref[1]
    h1 = jnp.maximum(agg * nd + b1_ref[...], 0.0)
    h2_ref[...] = jnp.dot(h1 * ns, w2_ref[...],
                          preferred_element_type=jnp.float32)


def _tc3_body(part_ref, norms_ref, b2_ref, wp_ref, bp_ref, seg_ref, cam_ref):
    nd = norms_ref[1].reshape(NPAD, 1)
    agg = part_ref[0] + part_ref[1]
    hidden = jnp.maximum(agg * nd + b2_ref[...], 0.0)[:N]
    wp = wp_ref[...]
    cam_ref[...] = lax.dot_general(wp, hidden, (((1,), (1,)), ((), ())),
                                   preferred_element_type=jnp.float32)
    hg = jnp.sum(hidden, axis=0, keepdims=True) * (1.0 / N)
    seg_ref[...] = lax.dot_general(hg, wp, (((1,), (1,)), ((), ())),
                                   preferred_element_type=jnp.float32) + bp_ref[...]


def _tc_call(body, out_shape, *args):
    return pl.pallas_call(body, out_shape=out_shape)(*args)


def kernel(features, edge_index, is_training, W1, b1, W2, b2, Wp, bp):
    del is_training
    e = edge_index.astype(jnp.int32)
    spread = N + jnp.arange(EP - E, dtype=jnp.int32) % (NPAD - N)
    pad = jnp.stack([spread, spread])
    e = jnp.concatenate([e, pad], axis=1).reshape(2, EROWS, 128)

    hist = _sc_degrees(e)
    norms, h1 = pl.pallas_call(
        _tc_h1_body,
        out_shape=(jax.ShapeDtypeStruct((2, NPAD), jnp.float32),
                   jax.ShapeDtypeStruct((NPAD, D), jnp.float32)),
    )(hist, features, W1)

    part1 = _sc_aggregate(h1, e)
    h2 = _tc_call(_tc2_body, jax.ShapeDtypeStruct((NPAD, D), jnp.float32),
                  part1, norms, b1.reshape(1, D), W2)
    part2 = _sc_aggregate(h2, e)
    seg, cam = pl.pallas_call(
        _tc3_body,
        out_shape=(jax.ShapeDtypeStruct((1, 4), jnp.float32),
                   jax.ShapeDtypeStruct((4, N), jnp.float32)),
    )(part2, norms, b2.reshape(1, D), Wp, bp.reshape(1, 4))
    return seg, cam
